# baseline (speedup 1.0000x reference)
import jax
import jax.numpy as jnp
from jax import lax
from jax.experimental import pallas as pl
from jax.experimental.pallas import tpu as pltpu
from jax.experimental.pallas import tpu_sc as plsc

_H = 128
_L = 16
_NV = _H // _L
_C = 128
_EPS = 1e-12


def _shuf(x, perm):
    dnums = lax.GatherDimensionNumbers(
        offset_dims=(), collapsed_slice_dims=(0,), start_index_map=(0,))
    return lax.gather(x, perm.reshape(_L, 1), dnums, (1,),
                      mode=lax.GatherScatterMode.PROMISE_IN_BOUNDS)


def _allsum(x, lanes):
    for k in (1, 2, 4, 8):
        x = x + _shuf(x, lanes ^ k)
    return x


def _rsqrt(v):
    i = lax.bitcast_convert_type(v, jnp.int32)
    i = jnp.int32(0x5F3759DF) - (i >> 1)
    y = lax.bitcast_convert_type(i, jnp.float32)
    h = 0.5 * v
    y = y * (1.5 - h * (y * y))
    return y


def _make_body(per_w, n_chunks):
    n_pairs = n_chunks // 2

    def _body(ids_h, pids_h, tids_h, wt_h, pt_h, small_h,
              out_h,
              widx, pidx, tidx,
              wrows0, prows0, obuf0, wrows1, prows1, obuf1, small_v, stat_v,
              semw0, semp0, semo0, semw1, semp1, semo1):
        c = lax.axis_index("c")
        s = lax.axis_index("s")
        wid = s * 2 + c
        base = wid * per_w
        lanes = lax.iota(jnp.int32, _L)

        slots = ((wrows0, prows0, obuf0, semw0, semp0, semo0),
                 (wrows1, prows1, obuf1, semw1, semp1, semo1))

        pltpu.sync_copy(ids_h.at[pl.ds(base, per_w)], widx)
        pltpu.sync_copy(pids_h.at[pl.ds(base, per_w)], pidx)
        pltpu.sync_copy(tids_h.at[pl.ds(base, per_w)], tidx)
        pltpu.sync_copy(small_h, small_v)
        g = [small_v[0, pl.ds(k * _L, _L)] for k in range(_NV)]
        b = [small_v[1, pl.ds(k * _L, _L)] for k in range(_NV)]
        t0 = [small_v[2, pl.ds(k * _L, _L)] for k in range(_NV)]
        dt = [small_v[3, pl.ds(k * _L, _L)] for k in range(_NV)]

        def issue_gathers(k, sl):
            wrows, prows = slots[sl][0], slots[sl][1]
            semw, semp = slots[sl][3], slots[sl][4]
            pltpu.async_copy(wt_h.at[widx.at[pl.ds(k * _C, _C)]], wrows, semw)
            pltpu.async_copy(pt_h.at[pidx.at[pl.ds(k * _C, _C)]], prows, semp)

        def wait_gathers(sl):
            wrows, prows = slots[sl][0], slots[sl][1]
            semw, semp = slots[sl][3], slots[sl][4]
            pltpu.make_async_copy(wt_h.at[widx.at[pl.ds(0, _C)]], wrows, semw).wait()
            pltpu.make_async_copy(pt_h.at[pidx.at[pl.ds(0, _C)]], prows, semp).wait()

        def issue_out(k, sl):
            obuf, semo = slots[sl][2], slots[sl][5]
            pltpu.async_copy(obuf, out_h.at[pl.ds(base + k * _C, _C)], semo)

        def wait_out(sl):
            obuf, semo = slots[sl][2], slots[sl][5]
            pltpu.make_async_copy(obuf, out_h.at[pl.ds(0, _C)], semo).wait()

        def compute(k, sl):
            wrows, prows, obuf = slots[sl][0], slots[sl][1], slots[sl][2]
            kbase = k * _C

            @plsc.parallel_loop(0, _C, unroll=5)
            def token(j):
                tgrp = tidx[pl.ds(kbase + (j & jnp.int32(~15)), _L)]
                tf = _shuf(tgrp, jnp.broadcast_to(j & 15, (_L,)))
                x = []
                for k2 in range(_NV):
                    x.append(wrows[j, pl.ds(k2 * _L, _L)]
                             + prows[j, pl.ds(k2 * _L, _L)]
                             + (t0[k2] + tf * dt[k2]))
                xs = ((x[0] + x[1]) + (x[2] + x[3])) + \
                     ((x[4] + x[5]) + (x[6] + x[7]))
                qs = ((x[0] * x[0] + x[1] * x[1]) + (x[2] * x[2] + x[3] * x[3])) + \
                     ((x[4] * x[4] + x[5] * x[5]) + (x[6] * x[6] + x[7] * x[7]))
                mv = _allsum(xs, lanes) * (1.0 / _H)
                var = _allsum(qs, lanes) * (1.0 / _H) - mv * mv
                r = _rsqrt(var + _EPS)
                mr = mv * r
                for k2 in range(_NV):
                    a = r * g[k2]
                    obuf[j, pl.ds(k2 * _L, _L)] = x[k2] * a + (b[k2] - mr * g[k2])

        issue_gathers(0, 0)

        def pair(i, carry):
            k0 = 2 * i
            k1 = k0 + 1
            wait_gathers(0)
            issue_gathers(k1, 1)
            pl.when(i >= 1)(lambda: wait_out(0))
            compute(k0, 0)
            issue_out(k0, 0)
            wait_gathers(1)
            pl.when(i < n_pairs - 1)(lambda: issue_gathers(k1 + 1, 0))
            pl.when(i >= 1)(lambda: wait_out(1))
            compute(k1, 1)
            issue_out(k1, 1)
            return carry

        lax.fori_loop(0, n_pairs, pair, 0)
        wait_out(0)
        wait_out(1)

    return _body


def kernel(input_ids, token_type_ids, position_ids, word_table, pos_table,
           type_table, ln_gamma, ln_beta):
    bsz, seq = input_ids.shape
    n = bsz * seq
    info = plsc.get_sparse_core_info()
    nw = info.num_cores * info.num_subcores
    per_w = n // nw
    n_chunks = per_w // _C
    assert per_w * nw == n and n_chunks * _C == per_w and n_chunks % 2 == 0

    ids = input_ids.reshape(n).astype(jnp.int32)
    pids = position_ids.reshape(n).astype(jnp.int32)
    tids = token_type_ids.reshape(n).astype(jnp.float32)
    small = jnp.stack([ln_gamma, ln_beta, type_table[0],
                       type_table[1] - type_table[0]])

    mesh = plsc.VectorSubcoreMesh(core_axis_name="c", subcore_axis_name="s")
    f = pl.kernel(
        _make_body(per_w, n_chunks),
        mesh=mesh,
        out_type=jax.ShapeDtypeStruct((n, _H), jnp.float32),
        scratch_types=[
            pltpu.VMEM((per_w,), jnp.int32),
            pltpu.VMEM((per_w,), jnp.int32),
            pltpu.VMEM((per_w,), jnp.float32),
            pltpu.VMEM((_C, _H), jnp.float32),
            pltpu.VMEM((_C, _H), jnp.float32),
            pltpu.VMEM((_C, _H), jnp.float32),
            pltpu.VMEM((_C, _H), jnp.float32),
            pltpu.VMEM((_C, _H), jnp.float32),
            pltpu.VMEM((_C, _H), jnp.float32),
            pltpu.VMEM((4, _H), jnp.float32),
            pltpu.VMEM((_C * 2 * _L,), jnp.float32),
            pltpu.SemaphoreType.DMA,
            pltpu.SemaphoreType.DMA,
            pltpu.SemaphoreType.DMA,
            pltpu.SemaphoreType.DMA,
            pltpu.SemaphoreType.DMA,
            pltpu.SemaphoreType.DMA,
        ],
    )
    out = f(ids, pids, tids, word_table, pos_table, small)
    return out.reshape(bsz, seq, _H)


# speedup vs baseline: 1.1113x; 1.1113x over previous
import jax
import jax.numpy as jnp
from jax import lax
from jax.experimental import pallas as pl
from jax.experimental.pallas import tpu as pltpu
from jax.experimental.pallas import tpu_sc as plsc

_H = 128
_L = 16
_NV = _H // _L
_C = 128
_EPS = 1e-12


def _shuf(x, perm):
    dnums = lax.GatherDimensionNumbers(
        offset_dims=(), collapsed_slice_dims=(0,), start_index_map=(0,))
    return lax.gather(x, perm.reshape(_L, 1), dnums, (1,),
                      mode=lax.GatherScatterMode.PROMISE_IN_BOUNDS)


def _rsqrt(v):
    i = lax.bitcast_convert_type(v, jnp.int32)
    i = jnp.int32(0x5F3759DF) - (i >> 1)
    y = lax.bitcast_convert_type(i, jnp.float32)
    h = 0.5 * v
    y = y * (1.5 - h * (y * y))
    return y


def _make_body(per_w, n_chunks):
    n_pairs = n_chunks // 2

    def _body(ids_h, pids_h, tids_h, wt_h, pt_h, small_h,
              out_h,
              widx, pidx, tidx,
              wrows0, prows0, obuf0, wrows1, prows1, obuf1, small_v,
              semw0, semp0, semo0, semw1, semp1, semo1):
        c = lax.axis_index("c")
        s = lax.axis_index("s")
        wid = s * 2 + c
        base = wid * per_w

        slots = ((wrows0, prows0, obuf0, semw0, semp0, semo0),
                 (wrows1, prows1, obuf1, semw1, semp1, semo1))

        pltpu.sync_copy(ids_h.at[pl.ds(base, per_w)], widx)
        pltpu.sync_copy(pids_h.at[pl.ds(base, per_w)], pidx)
        pltpu.sync_copy(tids_h.at[pl.ds(base, per_w)], tidx)
        pltpu.sync_copy(small_h, small_v)
        g = [small_v[0, pl.ds(k * _L, _L)] for k in range(_NV)]
        b = [small_v[1, pl.ds(k * _L, _L)] for k in range(_NV)]
        t0 = [small_v[2, pl.ds(k * _L, _L)] for k in range(_NV)]
        dt = [small_v[3, pl.ds(k * _L, _L)] for k in range(_NV)]

        def issue_gathers(k, sl):
            wrows, prows = slots[sl][0], slots[sl][1]
            semw, semp = slots[sl][3], slots[sl][4]
            pltpu.async_copy(wt_h.at[widx.at[pl.ds(k * _C, _C)]], wrows, semw)
            pltpu.async_copy(pt_h.at[pidx.at[pl.ds(k * _C, _C)]], prows, semp)

        def wait_gathers(sl):
            wrows, prows = slots[sl][0], slots[sl][1]
            semw, semp = slots[sl][3], slots[sl][4]
            pltpu.make_async_copy(wt_h.at[widx.at[pl.ds(0, _C)]], wrows, semw).wait()
            pltpu.make_async_copy(pt_h.at[pidx.at[pl.ds(0, _C)]], prows, semp).wait()

        def issue_out(k, sl):
            obuf, semo = slots[sl][2], slots[sl][5]
            pltpu.async_copy(obuf, out_h.at[pl.ds(base + k * _C, _C)], semo)

        def wait_out(sl):
            obuf, semo = slots[sl][2], slots[sl][5]
            pltpu.make_async_copy(obuf, out_h.at[pl.ds(0, _C)], semo).wait()

        def compute(k, sl):
            wrows, prows, obuf = slots[sl][0], slots[sl][1], slots[sl][2]
            kbase = k * _C

            @plsc.parallel_loop(0, _C, unroll=4)
            def token(j):
                tgrp = tidx[pl.ds(kbase + (j & jnp.int32(~15)), _L)]
                tf = _shuf(tgrp, jnp.broadcast_to(j & 15, (_L,)))
                x = []
                for k2 in range(_NV):
                    x.append(wrows[j, pl.ds(k2 * _L, _L)]
                             + prows[j, pl.ds(k2 * _L, _L)]
                             + (t0[k2] + tf * dt[k2]))
                xs = ((x[0] + x[1]) + (x[2] + x[3])) + \
                     ((x[4] + x[5]) + (x[6] + x[7]))
                qs = ((x[0] * x[0] + x[1] * x[1]) + (x[2] * x[2] + x[3] * x[3])) + \
                     ((x[4] * x[4] + x[5] * x[5]) + (x[6] * x[6] + x[7] * x[7]))
                mv = jnp.broadcast_to(jnp.sum(xs) * (1.0 / _H), (_L,))
                var = jnp.broadcast_to(jnp.sum(qs) * (1.0 / _H), (_L,)) - mv * mv
                r = _rsqrt(var + _EPS)
                mr = mv * r
                for k2 in range(_NV):
                    a = r * g[k2]
                    obuf[j, pl.ds(k2 * _L, _L)] = x[k2] * a + (b[k2] - mr * g[k2])

        issue_gathers(0, 0)

        def pair(i, carry):
            k0 = 2 * i
            k1 = k0 + 1
            wait_gathers(0)
            issue_gathers(k1, 1)
            pl.when(i >= 1)(lambda: wait_out(0))
            compute(k0, 0)
            issue_out(k0, 0)
            wait_gathers(1)
            pl.when(i < n_pairs - 1)(lambda: issue_gathers(k1 + 1, 0))
            pl.when(i >= 1)(lambda: wait_out(1))
            compute(k1, 1)
            issue_out(k1, 1)
            return carry

        lax.fori_loop(0, n_pairs, pair, 0)
        wait_out(0)
        wait_out(1)

    return _body


def kernel(input_ids, token_type_ids, position_ids, word_table, pos_table,
           type_table, ln_gamma, ln_beta):
    bsz, seq = input_ids.shape
    n = bsz * seq
    info = plsc.get_sparse_core_info()
    nw = info.num_cores * info.num_subcores
    per_w = n // nw
    n_chunks = per_w // _C
    assert per_w * nw == n and n_chunks * _C == per_w and n_chunks % 2 == 0

    ids = input_ids.reshape(n).astype(jnp.int32)
    pids = position_ids.reshape(n).astype(jnp.int32)
    tids = token_type_ids.reshape(n).astype(jnp.float32)
    small = jnp.stack([ln_gamma, ln_beta, type_table[0],
                       type_table[1] - type_table[0]])

    mesh = plsc.VectorSubcoreMesh(core_axis_name="c", subcore_axis_name="s")
    f = pl.kernel(
        _make_body(per_w, n_chunks),
        mesh=mesh,
        compiler_params=pltpu.CompilerParams(needs_layout_passes=False),
        out_type=jax.ShapeDtypeStruct((n, _H), jnp.float32),
        scratch_types=[
            pltpu.VMEM((per_w,), jnp.int32),
            pltpu.VMEM((per_w,), jnp.int32),
            pltpu.VMEM((per_w,), jnp.float32),
            pltpu.VMEM((_C, _H), jnp.float32),
            pltpu.VMEM((_C, _H), jnp.float32),
            pltpu.VMEM((_C, _H), jnp.float32),
            pltpu.VMEM((_C, _H), jnp.float32),
            pltpu.VMEM((_C, _H), jnp.float32),
            pltpu.VMEM((_C, _H), jnp.float32),
            pltpu.VMEM((4, _H), jnp.float32),
            pltpu.SemaphoreType.DMA,
            pltpu.SemaphoreType.DMA,
            pltpu.SemaphoreType.DMA,
            pltpu.SemaphoreType.DMA,
            pltpu.SemaphoreType.DMA,
            pltpu.SemaphoreType.DMA,
        ],
    )
    out = f(ids, pids, tids, word_table, pos_table, small)
    return out.reshape(bsz, seq, _H)
